# baseline (speedup 1.0000x reference)
import functools

import jax
import jax.numpy as jnp
from jax import lax
from jax.experimental import pallas as pl
from jax.experimental.pallas import tpu as pltpu
from jax.experimental.pallas import tpu_sc as plsc

D_MODEL = 64
SCALE = 8.0

NC = 2
NS = 16
NW = NC * NS
LANES = 16

CHUNK = 128


def _sc_embed(x2d, table, n_rows):
    rows_per_w = n_rows // NW
    b_total = n_rows * CHUNK
    mesh = plsc.VectorSubcoreMesh(core_axis_name="c", subcore_axis_name="s")

    @functools.partial(
        pl.kernel,
        out_type=jax.ShapeDtypeStruct((b_total, D_MODEL), jnp.float32),
        mesh=mesh,
        scratch_types=[
            pltpu.VMEM((rows_per_w, CHUNK), jnp.int32),
            pltpu.VMEM((CHUNK, D_MODEL), jnp.float32),
            pltpu.SemaphoreType.DMA,
        ],
        compiler_params=pltpu.CompilerParams(use_tc_tiling_on_sc=False),
    )
    def k(x_hbm, table_hbm, out_hbm, idx_v, rows_v, sem):
        wid = lax.axis_index("s") * NC + lax.axis_index("c")
        base_row = wid * rows_per_w
        pltpu.sync_copy(x_hbm.at[pl.ds(base_row, rows_per_w)], idx_v)

        @pl.loop(0, rows_per_w)
        def _chunk(c):
            pltpu.async_copy(table_hbm.at[idx_v.at[c]], rows_v, sem).wait()

            @pl.loop(0, CHUNK)
            def _row(r):
                for d in range(D_MODEL // LANES):
                    sl = pl.ds(d * LANES, LANES)
                    rows_v[r, sl] = rows_v[r, sl] * SCALE

            out_base = (base_row + c) * CHUNK
            pltpu.sync_copy(rows_v, out_hbm.at[pl.ds(out_base, CHUNK)])

    return k(x2d, table)


def kernel(x, table):
    b, s = x.shape
    total = b * s
    n_rows = total // CHUNK
    x2d = x.reshape(n_rows, CHUNK).astype(jnp.int32)
    out = _sc_embed(x2d, table, n_rows)
    return out.reshape(b, s, D_MODEL)
